# Initial kernel scaffold; baseline (speedup 1.0000x reference)
import jax
import jax.numpy as jnp
from jax.experimental import pallas as pl


def kernel(input, weight, weight_factor):
    raise NotImplementedError("write your pallas kernel here")


# speedup vs baseline: 11.3068x; 11.3068x over previous
import jax
import jax.numpy as jnp
from jax.experimental import pallas as pl
from jax.experimental.pallas import tpu as pltpu

_H = 512
_W = 512


def _stencil_body(w_ref, wf_ref, x_ref, o_ref):
    wf = jnp.clip(wf_ref[0, 0], 1.001, 254.999)
    wr = []
    for k in range(9):
        y = jnp.clip(w_ref[0, k], 0.001, 0.999) * wf
        wr.append(jnp.maximum(jnp.round(y), 0.001))

    x = x_ref[0]
    xp = jnp.pad(x, ((1, 1), (1, 1)))
    acc = wr[0] * jax.lax.slice(xp, (0, 0), (_H, _W))
    for idx in range(1, 9):
        i, j = divmod(idx, 3)
        acc = acc + wr[idx] * jax.lax.slice(xp, (i, j), (i + _H, j + _W))
    o_ref[0] = acc


def kernel(input, weight, weight_factor):
    b = input.shape[0]
    x = input.reshape(b, _H, _W)
    out = pl.pallas_call(
        _stencil_body,
        grid=(b,),
        in_specs=[
            pl.BlockSpec(memory_space=pltpu.SMEM),
            pl.BlockSpec(memory_space=pltpu.SMEM),
            pl.BlockSpec((1, _H, _W), lambda i: (i, 0, 0)),
        ],
        out_specs=pl.BlockSpec((1, _H, _W), lambda i: (i, 0, 0)),
        out_shape=jax.ShapeDtypeStruct((b, _H, _W), jnp.float32),
        compiler_params=pltpu.CompilerParams(
            dimension_semantics=("arbitrary",),
        ),
    )(weight, weight_factor, x)
    return out.reshape(1, b, 1, _H, _W)


# speedup vs baseline: 21.4388x; 1.8961x over previous
import jax
import jax.numpy as jnp
from jax.experimental import pallas as pl
from jax.experimental.pallas import tpu as pltpu

_H = 512
_W = 512


def _stencil_body(w_ref, wf_ref, x_ref, o_ref):
    wf = jnp.clip(wf_ref[0, 0], 1.001, 254.999)
    wr = []
    for k in range(9):
        y = jnp.clip(w_ref[0, k], 0.001, 0.999) * wf
        wr.append(jnp.maximum(jnp.round(y), 0.001))

    x = x_ref[0]
    zcol = jnp.zeros((_H, 1), jnp.float32)
    zrow = jnp.zeros((1, _W), jnp.float32)
    xl = jnp.concatenate([zcol, x[:, : _W - 1]], axis=1)
    xr = jnp.concatenate([x[:, 1:], zcol], axis=1)
    a0 = wr[0] * xl + wr[1] * x + wr[2] * xr
    a1 = wr[3] * xl + wr[4] * x + wr[5] * xr
    a2 = wr[6] * xl + wr[7] * x + wr[8] * xr
    o_ref[0] = (
        a1
        + jnp.concatenate([zrow, a0[: _H - 1]], axis=0)
        + jnp.concatenate([a2[1:], zrow], axis=0)
    )


def kernel(input, weight, weight_factor):
    b = input.shape[0]
    x = input.reshape(b, _H, _W)
    out = pl.pallas_call(
        _stencil_body,
        grid=(b,),
        in_specs=[
            pl.BlockSpec(memory_space=pltpu.SMEM),
            pl.BlockSpec(memory_space=pltpu.SMEM),
            pl.BlockSpec((1, _H, _W), lambda i: (i, 0, 0)),
        ],
        out_specs=pl.BlockSpec((1, _H, _W), lambda i: (i, 0, 0)),
        out_shape=jax.ShapeDtypeStruct((b, _H, _W), jnp.float32),
        compiler_params=pltpu.CompilerParams(
            dimension_semantics=("arbitrary",),
        ),
    )(weight, weight_factor, x)
    return out.reshape(1, b, 1, _H, _W)


# speedup vs baseline: 25.0464x; 1.1683x over previous
import jax
import jax.numpy as jnp
from jax.experimental import pallas as pl
from jax.experimental.pallas import tpu as pltpu

_H = 512
_W = 512


def _stencil_body(w_ref, wf_ref, x_ref, o_ref):
    wf = jnp.clip(wf_ref[0, 0], 1.001, 254.999)
    wr = []
    for k in range(9):
        y = jnp.clip(w_ref[0, k], 0.001, 0.999) * wf
        wr.append(jnp.maximum(jnp.round(y), 0.001))

    x = x_ref[...]
    nb = x.shape[0]
    zcol = jnp.zeros((nb, _H, 1), jnp.float32)
    zrow = jnp.zeros((nb, 1, _W), jnp.float32)
    xl = jnp.concatenate([zcol, x[:, :, : _W - 1]], axis=2)
    xr = jnp.concatenate([x[:, :, 1:], zcol], axis=2)
    a0 = wr[0] * xl + wr[1] * x + wr[2] * xr
    a1 = wr[3] * xl + wr[4] * x + wr[5] * xr
    a2 = wr[6] * xl + wr[7] * x + wr[8] * xr
    o_ref[...] = (
        a1
        + jnp.concatenate([zrow, a0[:, : _H - 1]], axis=1)
        + jnp.concatenate([a2[:, 1:], zrow], axis=1)
    )


def kernel(input, weight, weight_factor):
    b = input.shape[0]
    nb = 2
    x = input.reshape(b, _H, _W)
    out = pl.pallas_call(
        _stencil_body,
        grid=(b // nb,),
        in_specs=[
            pl.BlockSpec(memory_space=pltpu.SMEM),
            pl.BlockSpec(memory_space=pltpu.SMEM),
            pl.BlockSpec((nb, _H, _W), lambda i: (i, 0, 0)),
        ],
        out_specs=pl.BlockSpec((nb, _H, _W), lambda i: (i, 0, 0)),
        out_shape=jax.ShapeDtypeStruct((b, _H, _W), jnp.float32),
        compiler_params=pltpu.CompilerParams(
            dimension_semantics=("parallel",),
        ),
    )(weight, weight_factor, x)
    return out.reshape(1, b, 1, _H, _W)


# speedup vs baseline: 25.7422x; 1.0278x over previous
import jax
import jax.numpy as jnp
from jax.experimental import pallas as pl
from jax.experimental.pallas import tpu as pltpu

_H = 512
_W = 512


def _stencil_body(w_ref, wf_ref, x_ref, o_ref):
    wf = jnp.clip(wf_ref[0, 0], 1.001, 254.999)
    wr = []
    for k in range(9):
        y = jnp.clip(w_ref[0, k], 0.001, 0.999) * wf
        wr.append(jnp.maximum(jnp.round(y), 0.001))

    x = x_ref[...]
    nb = x.shape[0]
    zcol = jnp.zeros((nb, _H, 1), jnp.float32)
    zrow = jnp.zeros((nb, 1, _W), jnp.float32)
    xl = jnp.concatenate([zcol, x[:, :, : _W - 1]], axis=2)
    xr = jnp.concatenate([x[:, :, 1:], zcol], axis=2)
    a0 = wr[0] * xl + wr[1] * x + wr[2] * xr
    a1 = wr[3] * xl + wr[4] * x + wr[5] * xr
    a2 = wr[6] * xl + wr[7] * x + wr[8] * xr
    o_ref[...] = (
        a1
        + jnp.concatenate([zrow, a0[:, : _H - 1]], axis=1)
        + jnp.concatenate([a2[:, 1:], zrow], axis=1)
    )


def kernel(input, weight, weight_factor):
    b = input.shape[0]
    nb = 4
    x = input.reshape(b, _H, _W)
    out = pl.pallas_call(
        _stencil_body,
        grid=(b // nb,),
        in_specs=[
            pl.BlockSpec(memory_space=pltpu.SMEM),
            pl.BlockSpec(memory_space=pltpu.SMEM),
            pl.BlockSpec((nb, _H, _W), lambda i: (i, 0, 0)),
        ],
        out_specs=pl.BlockSpec((nb, _H, _W), lambda i: (i, 0, 0)),
        out_shape=jax.ShapeDtypeStruct((b, _H, _W), jnp.float32),
        compiler_params=pltpu.CompilerParams(
            dimension_semantics=("parallel",),
        ),
    )(weight, weight_factor, x)
    return out.reshape(1, b, 1, _H, _W)
